# baseline (speedup 1.0000x reference)
import functools

import jax
import jax.numpy as jnp
from jax import lax
from jax.experimental import pallas as pl
from jax.experimental.pallas import tpu as pltpu
from jax.experimental.pallas import tpu_sc as plsc

VOCAB = 1000000
EMB = 64
NCLASS = 9
BATCH = 16384
WIN = 5

NC = 2
NS = 16
NW = NC * NS

TOTAL_ROWS = BATCH * WIN
ROWS_PER_W = TOTAL_ROWS // NW
CHUNK = 128
NCHUNK = ROWS_PER_W // CHUNK


def _sc_gather_body(table_hbm, idx_hbm, out_hbm, idx_v, rows_a, rows_b, sem_a, sem_b):
    wid = lax.axis_index("s") * NC + lax.axis_index("c")
    base = wid * ROWS_PER_W
    pltpu.sync_copy(idx_hbm.at[wid], idx_v)

    pltpu.async_copy(table_hbm.at[idx_v.at[0]], rows_a, sem_a)

    def step(j, carry):
        del carry
        cp1 = pltpu.async_copy(table_hbm.at[idx_v.at[j + 1]], rows_b, sem_b)
        pltpu.make_async_copy(table_hbm.at[idx_v.at[0]], rows_a, sem_a).wait()
        pltpu.sync_copy(rows_a, out_hbm.at[pl.ds(base + j * CHUNK, CHUNK)])

        @pl.when(j + 2 < NCHUNK)
        def _():
            pltpu.async_copy(table_hbm.at[idx_v.at[j + 2]], rows_a, sem_a)

        cp1.wait()
        pltpu.sync_copy(rows_b, out_hbm.at[pl.ds(base + (j + 1) * CHUNK, CHUNK)])
        return 0

    lax.fori_loop(0, NCHUNK // 2, lambda s, c: step(2 * s, c), 0)


def _sc_gather(table, idx3):
    k = pl.kernel(
        _sc_gather_body,
        out_type=jax.ShapeDtypeStruct((TOTAL_ROWS, EMB), jnp.float32),
        mesh=plsc.VectorSubcoreMesh(core_axis_name="c", subcore_axis_name="s"),
        scratch_types=[
            pltpu.VMEM((NCHUNK, CHUNK), jnp.int32),
            pltpu.VMEM((CHUNK, EMB), jnp.float32),
            pltpu.VMEM((CHUNK, EMB), jnp.float32),
            pltpu.SemaphoreType.DMA,
            pltpu.SemaphoreType.DMA,
        ],
        compiler_params=pltpu.CompilerParams(use_tc_tiling_on_sc=False),
    )
    return k(table, idx3)


def _tc_matmul_body(x_ref, wt_ref, b_ref, o_ref):
    o_ref[...] = (
        jnp.dot(x_ref[...], wt_ref[...], preferred_element_type=jnp.float32)
        + b_ref[...]
    )


def _tc_matmul(xmat, wt, b2):
    blk = 2048
    grid = BATCH // blk
    return pl.pallas_call(
        _tc_matmul_body,
        grid=(grid,),
        in_specs=[
            pl.BlockSpec((blk, WIN * EMB), lambda i: (i, 0)),
            pl.BlockSpec((WIN * EMB, NCLASS), lambda i: (0, 0)),
            pl.BlockSpec((1, NCLASS), lambda i: (0, 0)),
        ],
        out_specs=pl.BlockSpec((blk, NCLASS), lambda i: (i, 0)),
        out_shape=jax.ShapeDtypeStruct((BATCH, NCLASS), jnp.float32),
    )(xmat, wt, b2)


@jax.jit
def kernel(x, table, W, b):
    idx3 = x.reshape(NW, NCHUNK, CHUNK)
    rows = _sc_gather(table, idx3)
    xmat = rows.reshape(BATCH, WIN * EMB)
    return _tc_matmul(xmat, W.T, b.reshape(1, NCLASS))
